# baseline (speedup 1.0000x reference)
import functools

import jax
import jax.numpy as jnp
from jax import lax
from jax.experimental import pallas as pl
from jax.experimental.pallas import tpu as pltpu
from jax.experimental.pallas import tpu_sc as plsc

NN = 10000
EE = 160000
DH = 128
NL = 3

NR = 10112
JUNK_ROW = 10008
CH = 80
IB = 8
CPB = IB * CH
NBLK = 17
NCHUNK = NBLK * IB
TPT = NCHUNK * CH
EP = 16 * TPT
RPT = NR // 16
RB = 1000


def _iotas():
    ri = lax.broadcasted_iota(jnp.int32, (DH, DH), 0)
    ci = lax.broadcasted_iota(jnp.int32, (DH, DH), 1)
    return ri, ci


def _perm_mats():
    ri, ci = _iotas()
    cc = ci % 32
    k = ci // 32
    feat = cc < 16
    p0 = (feat & (ri == 16 * k + cc)).astype(jnp.float32)
    p1 = (feat & (ri == 64 + 16 * k + cc)).astype(jnp.float32)
    return p0, p1


def _den_mat():
    ri, ci = _iotas()
    cc = ci % 32
    k = ci // 32
    return ((cc < 16) & (ri == 32 * k + 16 + cc)).astype(jnp.float32)


def _sel_mats():
    ri, ci = _iotas()
    rr = ri % 32
    k = ri // 32
    s0 = ((rr < 16) & (ci == 16 * k + rr)).astype(jnp.float32)
    s1 = ((rr < 16) & (ci == 64 + 16 * k + rr)).astype(jnp.float32)
    return s0, s1


def _hist_body(ea_ref, cnt_ref):
    ea = ea_ref[...]
    for t in range(3):
        cnt_ref[t] = jnp.sum((ea == t).astype(jnp.int32))


def _edge_attr_hist(ea2d):
    return pl.pallas_call(
        _hist_body,
        out_shape=jax.ShapeDtypeStruct((3,), jnp.int32),
        in_specs=[pl.BlockSpec(memory_space=pltpu.VMEM)],
        out_specs=pl.BlockSpec(memory_space=pltpu.SMEM),
    )(ea2d)


def _eetab_body(etp_ref, cnt_ref, we_ref, out_ref):
    etv = etp_ref[...]
    w = [cnt_ref[t].astype(jnp.float32) / float(EE) for t in range(3)]
    mean = w[0] * etv[0:1, :] + w[1] * etv[1:2, :] + w[2] * etv[2:3, :]
    rows = lax.broadcasted_iota(jnp.int32, (8, DH), 0)
    et_in = jnp.where(rows == 3, jnp.broadcast_to(mean, (8, DH)), etv)
    e = jnp.dot(et_in, we_ref[0], preferred_element_type=jnp.float32)
    p0, p1 = _perm_mats()
    e0 = jnp.dot(e, p0, preferred_element_type=jnp.float32)
    e1 = jnp.dot(e, p1, preferred_element_type=jnp.float32)
    out_ref[0] = jnp.concatenate([e0[0:4], e1[0:4]], axis=0)


def _eetab(etp, counts, We):
    return pl.pallas_call(
        _eetab_body,
        grid=(NL,),
        out_shape=jax.ShapeDtypeStruct((NL, 8, DH), jnp.float32),
        in_specs=[
            pl.BlockSpec((8, DH), lambda l: (0, 0)),
            pl.BlockSpec(memory_space=pltpu.SMEM),
            pl.BlockSpec((1, DH, DH), lambda l: (l, 0, 0)),
        ],
        out_specs=pl.BlockSpec((1, 8, DH), lambda l: (l, 0, 0)),
    )(etp, counts, We)


def _project(h, wl_ref, bl_ref, wr_ref, br_ref, xlT_ref, xrT_ref):
    p0, p1 = _perm_mats()
    xl = jnp.dot(h, wl_ref[...], preferred_element_type=jnp.float32) + bl_ref[...]
    xr = jnp.dot(h, wr_ref[...], preferred_element_type=jnp.float32) + br_ref[...]
    xlT_ref[0] = jnp.dot(xl, p0, preferred_element_type=jnp.float32)
    xlT_ref[1] = jnp.dot(xl, p1, preferred_element_type=jnp.float32)
    xrT_ref[0] = jnp.dot(xr, p0, preferred_element_type=jnp.float32)
    xrT_ref[1] = jnp.dot(xr, p1, preferred_element_type=jnp.float32)


def _prologue_body(x_ref, win_ref, bin_ref, ntoh_ref, ntt_ref,
                   wl_ref, bl_ref, wr_ref, br_ref,
                   h_ref, xlT_ref, xrT_ref):
    h = (jnp.dot(x_ref[...], win_ref[...], preferred_element_type=jnp.float32)
         + bin_ref[...]
         + jnp.dot(ntoh_ref[...], ntt_ref[...],
                   preferred_element_type=jnp.float32))
    h_ref[...] = h
    _project(h, wl_ref, bl_ref, wr_ref, br_ref, xlT_ref, xrT_ref)


def _prologue(x, W_in, b_in, ntoh, ntt, Wl0, bl0, Wr0, br0):
    full = lambda i: (0, 0)
    return pl.pallas_call(
        _prologue_body,
        grid=(NN // RB,),
        out_shape=[
            jax.ShapeDtypeStruct((NN, DH), jnp.float32),
            jax.ShapeDtypeStruct((2, NN, DH), jnp.float32),
            jax.ShapeDtypeStruct((2, NN, DH), jnp.float32),
        ],
        in_specs=[
            pl.BlockSpec((RB, DH), lambda i: (i, 0)),
            pl.BlockSpec((DH, DH), full),
            pl.BlockSpec((1, DH), full),
            pl.BlockSpec((RB, 8), lambda i: (i, 0)),
            pl.BlockSpec((8, DH), full),
            pl.BlockSpec((DH, DH), full),
            pl.BlockSpec((1, DH), full),
            pl.BlockSpec((DH, DH), full),
            pl.BlockSpec((1, DH), full),
        ],
        out_specs=[
            pl.BlockSpec((RB, DH), lambda i: (i, 0)),
            pl.BlockSpec((2, RB, DH), lambda i: (0, i, 0)),
            pl.BlockSpec((2, RB, DH), lambda i: (0, i, 0)),
        ],
    )(x, W_in, b_in, ntoh, ntt, Wl0, bl0, Wr0, br0)


_sc_mesh = plsc.VectorSubcoreMesh(core_axis_name="c", subcore_axis_name="s")


@functools.partial(
    pl.kernel,
    out_type=jax.ShapeDtypeStruct((2, NR, DH), jnp.float32),
    mesh=_sc_mesh,
    compiler_params=pltpu.CompilerParams(needs_layout_passes=False),
    scratch_types=[
        pltpu.VMEM((IB, CH), jnp.int32),
        pltpu.VMEM((IB, CH), jnp.int32),
        pltpu.VMEM((IB, CH), jnp.int32),
        pltpu.VMEM((CPB + 16,), jnp.int32),
        pltpu.VMEM((CH, DH), jnp.float32),
        pltpu.VMEM((CH, DH), jnp.float32),
        pltpu.VMEM((CH, DH), jnp.float32),
        pltpu.VMEM((CH, DH), jnp.float32),
        pltpu.VMEM((8, DH), jnp.float32),
        pltpu.VMEM((8, 16), jnp.float32),
        pltpu.VMEM_SHARED((NR, DH), jnp.float32),
        pltpu.SemaphoreType.DMA,
        pltpu.SemaphoreType.DMA,
        pltpu.SemaphoreType.DMA,
        pltpu.SemaphoreType.DMA,
    ],
)
def _sc_edge_kernel(xlT_hbm, xrT_hbm, src_hbm, dstg_hbm, dst_hbm, sel_hbm,
                    eetab_hbm, att_hbm, num_out,
                    isrc, idstg, idst, sel_v, xl0, xl1, xr0, xr1,
                    eetab_v, att_v, acc, sg0, sg1, ss0, ss1):
    cid = lax.axis_index("c")
    sid = lax.axis_index("s")
    row0 = sid * RPT
    cid4 = cid * 4
    gbase = cid * (EP // CH) + sid * (TPT // CH)
    rbase = sid * (TPT // CH)
    fbase = cid * EP + sid * TPT

    pltpu.sync_copy(eetab_hbm, eetab_v)
    pltpu.sync_copy(att_hbm, att_v)
    att_k = [att_v[k4, :] for k4 in range(4)]
    att_k = [jnp.where(cid4 > 0, att_v[4 + k4, :], att_k[k4])
             for k4 in range(4)]

    xlb = [xl0, xl1]
    xrb = [xr0, xr1]
    sg = [sg0, sg1]
    ss = [ss0, ss1]

    zl16 = jnp.zeros((16,), jnp.float32)

    def zrow(r, _):
        for h in range(8):
            xl0[r, pl.ds(h * 16, 16)] = zl16
        return 0

    lax.fori_loop(0, CH, zrow, 0)
    for k in range(RPT // CH):
        pltpu.sync_copy(xl0, acc.at[pl.ds(row0 + k * CH, CH)])
    tail = RPT - (RPT // CH) * CH
    if tail:
        pltpu.sync_copy(xl0.at[pl.ds(0, tail)],
                        acc.at[pl.ds(row0 + (RPT // CH) * CH, tail)])
    plsc.subcore_barrier()

    def run_edges(b, c):
        @plsc.parallel_loop(0, CH, unroll=4)
        def edge_body(j):
            s = sel_v[pl.ds(c * CH + j, 16)][0] + cid4
            for k4 in range(4):
                fsl = pl.ds(k4 * 32, 16)
                psl = pl.ds(k4 * 32 + 16, 16)
                xf = xlb[b][j, fsl]
                m = xf + xrb[b][j, fsl] + eetab_v[s, fsl]
                m = jnp.where(m >= 0.0, m, m * 0.2)
                a = jnp.sum(m * att_k[k4])
                pv = jnp.exp(jnp.broadcast_to(a, (16,)))
                xlb[b][j, fsl] = pv * xf
                xlb[b][j, psl] = pv

    def start_gather(c, b):
        g1 = pltpu.async_copy(xlT_hbm.at[isrc.at[c]], xlb[b], sg[b])
        g2 = pltpu.async_copy(xrT_hbm.at[idstg.at[c]], xrb[b], sg[b])
        return g1, g2

    def block_body(blk, _):
        pltpu.sync_copy(src_hbm.at[pl.ds(gbase + blk * IB, IB)], isrc)
        pltpu.sync_copy(dstg_hbm.at[pl.ds(gbase + blk * IB, IB)], idstg)
        pltpu.sync_copy(dst_hbm.at[pl.ds(rbase + blk * IB, IB)], idst)
        pltpu.sync_copy(sel_hbm.at[pl.ds(fbase + blk * CPB, CPB)],
                        sel_v.at[pl.ds(0, CPB)])
        gd = start_gather(0, 0)
        sd = [None, None]
        for c in range(IB):
            b = c % 2
            b1 = 1 - b
            if c + 1 < IB:
                if sd[b1] is not None:
                    sd[b1].wait()
                gd_next = start_gather(c + 1, b1)
            gd[0].wait()
            gd[1].wait()
            run_edges(b, c)
            sd[b] = pltpu.async_copy(xlb[b], acc.at[idst.at[c]], ss[b],
                                     add=True)
            if c + 1 < IB:
                gd = gd_next
        sd[0].wait()
        sd[1].wait()
        return 0

    lax.fori_loop(0, NBLK, block_body, 0)
    plsc.subcore_barrier()

    pltpu.sync_copy(acc.at[pl.ds(row0, RPT)],
                    num_out.at[cid, pl.ds(row0, RPT)])


def _normalize(num_ref, hprev_ref, bias_ref):
    nu0 = num_ref[0]
    nu1 = num_ref[1]
    d = _den_mat()
    s0, s1 = _sel_mats()
    der0 = jnp.dot(nu0, d, preferred_element_type=jnp.float32)
    der1 = jnp.dot(nu1, d, preferred_element_type=jnp.float32)
    r0 = nu0 / (der0 + 1e-16)
    r1 = nu1 / (der1 + 1e-16)
    o = (jnp.dot(r0, s0, preferred_element_type=jnp.float32)
         + jnp.dot(r1, s1, preferred_element_type=jnp.float32)
         + bias_ref[...])
    return jnp.maximum(o, 0.0) + hprev_ref[...]


def _combine_body(num_ref, hprev_ref, bias_ref,
                  wl_ref, bl_ref, wr_ref, br_ref,
                  h_ref, xlT_ref, xrT_ref):
    hn = _normalize(num_ref, hprev_ref, bias_ref)
    h_ref[...] = hn
    _project(hn, wl_ref, bl_ref, wr_ref, br_ref, xlT_ref, xrT_ref)


def _combine(num, hprev, bias_l, Wln, bln, Wrn, brn):
    full = lambda i: (0, 0)
    return pl.pallas_call(
        _combine_body,
        grid=(NN // RB,),
        out_shape=[
            jax.ShapeDtypeStruct((NN, DH), jnp.float32),
            jax.ShapeDtypeStruct((2, NN, DH), jnp.float32),
            jax.ShapeDtypeStruct((2, NN, DH), jnp.float32),
        ],
        in_specs=[
            pl.BlockSpec((2, RB, DH), lambda i: (0, i, 0)),
            pl.BlockSpec((RB, DH), lambda i: (i, 0)),
            pl.BlockSpec((1, DH), full),
            pl.BlockSpec((DH, DH), full),
            pl.BlockSpec((1, DH), full),
            pl.BlockSpec((DH, DH), full),
            pl.BlockSpec((1, DH), full),
        ],
        out_specs=[
            pl.BlockSpec((RB, DH), lambda i: (i, 0)),
            pl.BlockSpec((2, RB, DH), lambda i: (0, i, 0)),
            pl.BlockSpec((2, RB, DH), lambda i: (0, i, 0)),
        ],
    )(num, hprev, bias_l, Wln, bln, Wrn, brn)


def _final_body(num_ref, hprev_ref, bias_ref, wo_ref, bo_ref, out_ref):
    hn = _normalize(num_ref, hprev_ref, bias_ref)
    out_ref[...] = jnp.dot(hn, wo_ref[...],
                           preferred_element_type=jnp.float32) + bo_ref[...]


def _final(num, hprev, bias_l, W_out, b_out):
    full = lambda i: (0, 0)
    return pl.pallas_call(
        _final_body,
        grid=(NN // RB,),
        out_shape=jax.ShapeDtypeStruct((NN, DH), jnp.float32),
        in_specs=[
            pl.BlockSpec((2, RB, DH), lambda i: (0, i, 0)),
            pl.BlockSpec((RB, DH), lambda i: (i, 0)),
            pl.BlockSpec((1, DH), full),
            pl.BlockSpec((DH, DH), full),
            pl.BlockSpec((1, DH), full),
        ],
        out_specs=pl.BlockSpec((RB, DH), lambda i: (i, 0)),
    )(num, hprev, bias_l, W_out, b_out)


def kernel(x, edge_index, edge_attr, node_types, W_in, b_in, nt_table,
           et_table, Wl, bl, Wr, br, We, att, bias, W_out, b_out):
    loop = jnp.arange(NN, dtype=edge_index.dtype)
    npad = EP - (EE + NN)
    src2 = jnp.concatenate([edge_index[0], loop,
                            jnp.zeros((npad,), jnp.int32)])
    dst2 = jnp.concatenate([edge_index[1], loop,
                            jnp.full((npad,), JUNK_ROW, jnp.int32)])
    sel2 = jnp.concatenate([edge_attr, jnp.full((NN,), 3, jnp.int32),
                            jnp.zeros((npad,), jnp.int32)])
    dmin = jnp.minimum(dst2, NN - 1)
    src3 = jnp.concatenate([src2, src2 + NN]).reshape(2 * EP // CH, CH)
    dstg3 = jnp.concatenate([dmin, dmin + NN]).reshape(2 * EP // CH, CH)
    dst2d = dst2.reshape(EP // CH, CH)
    sel3 = jnp.concatenate([sel2, sel2 + 4])

    ntoh = (node_types[:, None] == jnp.arange(8)[None, :]).astype(jnp.float32)
    ntt8 = jnp.concatenate([nt_table, jnp.zeros((3, DH), jnp.float32)], axis=0)
    etp8 = jnp.concatenate([et_table, jnp.zeros((5, DH), jnp.float32)], axis=0)

    b_in2 = b_in.reshape(1, DH)
    bl2 = bl.reshape(NL, 1, DH)
    br2 = br.reshape(NL, 1, DH)
    bias2 = bias.reshape(NL, 1, DH)
    b_out2 = b_out.reshape(1, DH)

    counts = _edge_attr_hist(edge_attr.reshape(1250, 128))
    eetab = _eetab(etp8, counts, We)

    h, xlT, xrT = _prologue(x, W_in, b_in2, ntoh, ntt8,
                            Wl[0], bl2[0], Wr[0], br2[0])

    out = None
    for l in range(NL):
        num = _sc_edge_kernel(xlT.reshape(2 * NN, DH), xrT.reshape(2 * NN, DH),
                              src3, dstg3, dst2d, sel3, eetab[l], att[l])
        if l + 1 < NL:
            h, xlT, xrT = _combine(num, h, bias2[l],
                                   Wl[l + 1], bl2[l + 1], Wr[l + 1], br2[l + 1])
        else:
            out = _final(num, h, bias2[l], W_out, b_out2)
    return out


# speedup vs baseline: 1.0094x; 1.0094x over previous
import functools

import jax
import jax.numpy as jnp
from jax import lax
from jax.experimental import pallas as pl
from jax.experimental.pallas import tpu as pltpu
from jax.experimental.pallas import tpu_sc as plsc

NN = 10000
EE = 160000
DH = 128
NL = 3

NR = 10112
JUNK_ROW = 10008
CH = 64
IB = 8
CPB = IB * CH
NBLK = 21
NCHUNK = NBLK * IB
TPT = NCHUNK * CH
EP = 16 * TPT
RPT = NR // 16
RB = 1000


def _iotas():
    ri = lax.broadcasted_iota(jnp.int32, (DH, DH), 0)
    ci = lax.broadcasted_iota(jnp.int32, (DH, DH), 1)
    return ri, ci


def _perm_mats():
    ri, ci = _iotas()
    cc = ci % 32
    k = ci // 32
    feat = cc < 16
    p0 = (feat & (ri == 16 * k + cc)).astype(jnp.float32)
    p1 = (feat & (ri == 64 + 16 * k + cc)).astype(jnp.float32)
    return p0, p1


def _den_mat():
    ri, ci = _iotas()
    cc = ci % 32
    k = ci // 32
    return ((cc < 16) & (ri == 32 * k + 16 + cc)).astype(jnp.float32)


def _sel_mats():
    ri, ci = _iotas()
    rr = ri % 32
    k = ri // 32
    s0 = ((rr < 16) & (ci == 16 * k + rr)).astype(jnp.float32)
    s1 = ((rr < 16) & (ci == 64 + 16 * k + rr)).astype(jnp.float32)
    return s0, s1


def _hist_body(ea_ref, cnt_ref):
    ea = ea_ref[...]
    for t in range(3):
        cnt_ref[t] = jnp.sum((ea == t).astype(jnp.int32))


def _edge_attr_hist(ea2d):
    return pl.pallas_call(
        _hist_body,
        out_shape=jax.ShapeDtypeStruct((3,), jnp.int32),
        in_specs=[pl.BlockSpec(memory_space=pltpu.VMEM)],
        out_specs=pl.BlockSpec(memory_space=pltpu.SMEM),
    )(ea2d)


def _eetab_body(etp_ref, cnt_ref, we_ref, out_ref):
    etv = etp_ref[...]
    w = [cnt_ref[t].astype(jnp.float32) / float(EE) for t in range(3)]
    mean = w[0] * etv[0:1, :] + w[1] * etv[1:2, :] + w[2] * etv[2:3, :]
    rows = lax.broadcasted_iota(jnp.int32, (8, DH), 0)
    et_in = jnp.where(rows == 3, jnp.broadcast_to(mean, (8, DH)), etv)
    e = jnp.dot(et_in, we_ref[0], preferred_element_type=jnp.float32)
    p0, p1 = _perm_mats()
    e0 = jnp.dot(e, p0, preferred_element_type=jnp.float32)
    e1 = jnp.dot(e, p1, preferred_element_type=jnp.float32)
    out_ref[0] = jnp.concatenate([e0[0:4], e1[0:4]], axis=0)


def _eetab(etp, counts, We):
    return pl.pallas_call(
        _eetab_body,
        grid=(NL,),
        out_shape=jax.ShapeDtypeStruct((NL, 8, DH), jnp.float32),
        in_specs=[
            pl.BlockSpec((8, DH), lambda l: (0, 0)),
            pl.BlockSpec(memory_space=pltpu.SMEM),
            pl.BlockSpec((1, DH, DH), lambda l: (l, 0, 0)),
        ],
        out_specs=pl.BlockSpec((1, 8, DH), lambda l: (l, 0, 0)),
    )(etp, counts, We)


def _project(h, wl_ref, bl_ref, wr_ref, br_ref, xlT_ref, xrT_ref):
    p0, p1 = _perm_mats()
    xl = jnp.dot(h, wl_ref[...], preferred_element_type=jnp.float32) + bl_ref[...]
    xr = jnp.dot(h, wr_ref[...], preferred_element_type=jnp.float32) + br_ref[...]
    xlT_ref[0] = jnp.dot(xl, p0, preferred_element_type=jnp.float32)
    xlT_ref[1] = jnp.dot(xl, p1, preferred_element_type=jnp.float32)
    xrT_ref[0] = jnp.dot(xr, p0, preferred_element_type=jnp.float32)
    xrT_ref[1] = jnp.dot(xr, p1, preferred_element_type=jnp.float32)


def _prologue_body(x_ref, win_ref, bin_ref, ntoh_ref, ntt_ref,
                   wl_ref, bl_ref, wr_ref, br_ref,
                   h_ref, xlT_ref, xrT_ref):
    h = (jnp.dot(x_ref[...], win_ref[...], preferred_element_type=jnp.float32)
         + bin_ref[...]
         + jnp.dot(ntoh_ref[...], ntt_ref[...],
                   preferred_element_type=jnp.float32))
    h_ref[...] = h
    _project(h, wl_ref, bl_ref, wr_ref, br_ref, xlT_ref, xrT_ref)


def _prologue(x, W_in, b_in, ntoh, ntt, Wl0, bl0, Wr0, br0):
    full = lambda i: (0, 0)
    return pl.pallas_call(
        _prologue_body,
        grid=(NN // RB,),
        out_shape=[
            jax.ShapeDtypeStruct((NN, DH), jnp.float32),
            jax.ShapeDtypeStruct((2, NN, DH), jnp.float32),
            jax.ShapeDtypeStruct((2, NN, DH), jnp.float32),
        ],
        in_specs=[
            pl.BlockSpec((RB, DH), lambda i: (i, 0)),
            pl.BlockSpec((DH, DH), full),
            pl.BlockSpec((1, DH), full),
            pl.BlockSpec((RB, 8), lambda i: (i, 0)),
            pl.BlockSpec((8, DH), full),
            pl.BlockSpec((DH, DH), full),
            pl.BlockSpec((1, DH), full),
            pl.BlockSpec((DH, DH), full),
            pl.BlockSpec((1, DH), full),
        ],
        out_specs=[
            pl.BlockSpec((RB, DH), lambda i: (i, 0)),
            pl.BlockSpec((2, RB, DH), lambda i: (0, i, 0)),
            pl.BlockSpec((2, RB, DH), lambda i: (0, i, 0)),
        ],
    )(x, W_in, b_in, ntoh, ntt, Wl0, bl0, Wr0, br0)


_sc_mesh = plsc.VectorSubcoreMesh(core_axis_name="c", subcore_axis_name="s")


@functools.partial(
    pl.kernel,
    out_type=jax.ShapeDtypeStruct((2, NR, DH), jnp.float32),
    mesh=_sc_mesh,
    compiler_params=pltpu.CompilerParams(needs_layout_passes=False),
    scratch_types=[
        pltpu.VMEM((IB, CH), jnp.int32),
        pltpu.VMEM((IB, CH), jnp.int32),
        pltpu.VMEM((IB, CH), jnp.int32),
        pltpu.VMEM((CPB + 16,), jnp.int32),
        pltpu.VMEM((CH, DH), jnp.float32),
        pltpu.VMEM((CH, DH), jnp.float32),
        pltpu.VMEM((CH, DH), jnp.float32),
        pltpu.VMEM((CH, DH), jnp.float32),
        pltpu.VMEM((8, DH), jnp.float32),
        pltpu.VMEM((8, 16), jnp.float32),
        pltpu.VMEM_SHARED((NR, DH), jnp.float32),
        pltpu.SemaphoreType.DMA,
        pltpu.SemaphoreType.DMA,
        pltpu.SemaphoreType.DMA,
        pltpu.SemaphoreType.DMA,
    ],
)
def _sc_edge_kernel(xlT_hbm, xrT_hbm, src_hbm, dstg_hbm, dst_hbm, sel_hbm,
                    eetab_hbm, att_hbm, num_out,
                    isrc, idstg, idst, sel_v, xl0, xl1, xr0, xr1,
                    eetab_v, att_v, acc, sg0, sg1, ss0, ss1):
    cid = lax.axis_index("c")
    sid = lax.axis_index("s")
    row0 = sid * RPT
    cid4 = cid * 4
    gbase = cid * (EP // CH) + sid * (TPT // CH)
    rbase = sid * (TPT // CH)
    fbase = cid * EP + sid * TPT

    pltpu.sync_copy(eetab_hbm, eetab_v)
    pltpu.sync_copy(att_hbm, att_v)
    att_k = [att_v[k4, :] for k4 in range(4)]
    att_k = [jnp.where(cid4 > 0, att_v[4 + k4, :], att_k[k4])
             for k4 in range(4)]

    xlb = [xl0, xl1]
    xrb = [xr0, xr1]
    sg = [sg0, sg1]
    ss = [ss0, ss1]

    zl16 = jnp.zeros((16,), jnp.float32)

    def zrow(r, _):
        for h in range(8):
            xl0[r, pl.ds(h * 16, 16)] = zl16
        return 0

    lax.fori_loop(0, CH, zrow, 0)
    for k in range(RPT // CH):
        pltpu.sync_copy(xl0, acc.at[pl.ds(row0 + k * CH, CH)])
    tail = RPT - (RPT // CH) * CH
    if tail:
        pltpu.sync_copy(xl0.at[pl.ds(0, tail)],
                        acc.at[pl.ds(row0 + (RPT // CH) * CH, tail)])
    plsc.subcore_barrier()

    def run_edges(b, c):
        @plsc.parallel_loop(0, CH, unroll=6)
        def edge_body(j):
            s = sel_v[pl.ds(c * CH + j, 16)][0] + cid4
            for k4 in range(4):
                fsl = pl.ds(k4 * 32, 16)
                psl = pl.ds(k4 * 32 + 16, 16)
                xf = xlb[b][j, fsl]
                m = xf + xrb[b][j, fsl] + eetab_v[s, fsl]
                m = jnp.where(m >= 0.0, m, m * 0.2)
                a = jnp.sum(m * att_k[k4])
                pv = jnp.exp(jnp.broadcast_to(a, (16,)))
                xlb[b][j, fsl] = pv * xf
                xlb[b][j, psl] = pv

    def start_gather(c, b):
        g1 = pltpu.async_copy(xlT_hbm.at[isrc.at[c]], xlb[b], sg[b])
        g2 = pltpu.async_copy(xrT_hbm.at[idstg.at[c]], xrb[b], sg[b])
        return g1, g2

    def block_body(blk, _):
        pltpu.sync_copy(src_hbm.at[pl.ds(gbase + blk * IB, IB)], isrc)
        pltpu.sync_copy(dstg_hbm.at[pl.ds(gbase + blk * IB, IB)], idstg)
        pltpu.sync_copy(dst_hbm.at[pl.ds(rbase + blk * IB, IB)], idst)
        pltpu.sync_copy(sel_hbm.at[pl.ds(fbase + blk * CPB, CPB)],
                        sel_v.at[pl.ds(0, CPB)])
        gd = start_gather(0, 0)
        sd = [None, None]
        for c in range(IB):
            b = c % 2
            b1 = 1 - b
            if c + 1 < IB:
                if sd[b1] is not None:
                    sd[b1].wait()
                gd_next = start_gather(c + 1, b1)
            gd[0].wait()
            gd[1].wait()
            run_edges(b, c)
            sd[b] = pltpu.async_copy(xlb[b], acc.at[idst.at[c]], ss[b],
                                     add=True)
            if c + 1 < IB:
                gd = gd_next
        sd[0].wait()
        sd[1].wait()
        return 0

    lax.fori_loop(0, NBLK, block_body, 0)
    plsc.subcore_barrier()

    pltpu.sync_copy(acc.at[pl.ds(row0, RPT)],
                    num_out.at[cid, pl.ds(row0, RPT)])


def _normalize(num_ref, hprev_ref, bias_ref):
    nu0 = num_ref[0]
    nu1 = num_ref[1]
    d = _den_mat()
    s0, s1 = _sel_mats()
    der0 = jnp.dot(nu0, d, preferred_element_type=jnp.float32)
    der1 = jnp.dot(nu1, d, preferred_element_type=jnp.float32)
    r0 = nu0 / (der0 + 1e-16)
    r1 = nu1 / (der1 + 1e-16)
    o = (jnp.dot(r0, s0, preferred_element_type=jnp.float32)
         + jnp.dot(r1, s1, preferred_element_type=jnp.float32)
         + bias_ref[...])
    return jnp.maximum(o, 0.0) + hprev_ref[...]


def _combine_body(num_ref, hprev_ref, bias_ref,
                  wl_ref, bl_ref, wr_ref, br_ref,
                  h_ref, xlT_ref, xrT_ref):
    hn = _normalize(num_ref, hprev_ref, bias_ref)
    h_ref[...] = hn
    _project(hn, wl_ref, bl_ref, wr_ref, br_ref, xlT_ref, xrT_ref)


def _combine(num, hprev, bias_l, Wln, bln, Wrn, brn):
    full = lambda i: (0, 0)
    return pl.pallas_call(
        _combine_body,
        grid=(NN // RB,),
        out_shape=[
            jax.ShapeDtypeStruct((NN, DH), jnp.float32),
            jax.ShapeDtypeStruct((2, NN, DH), jnp.float32),
            jax.ShapeDtypeStruct((2, NN, DH), jnp.float32),
        ],
        in_specs=[
            pl.BlockSpec((2, RB, DH), lambda i: (0, i, 0)),
            pl.BlockSpec((RB, DH), lambda i: (i, 0)),
            pl.BlockSpec((1, DH), full),
            pl.BlockSpec((DH, DH), full),
            pl.BlockSpec((1, DH), full),
            pl.BlockSpec((DH, DH), full),
            pl.BlockSpec((1, DH), full),
        ],
        out_specs=[
            pl.BlockSpec((RB, DH), lambda i: (i, 0)),
            pl.BlockSpec((2, RB, DH), lambda i: (0, i, 0)),
            pl.BlockSpec((2, RB, DH), lambda i: (0, i, 0)),
        ],
    )(num, hprev, bias_l, Wln, bln, Wrn, brn)


def _final_body(num_ref, hprev_ref, bias_ref, wo_ref, bo_ref, out_ref):
    hn = _normalize(num_ref, hprev_ref, bias_ref)
    out_ref[...] = jnp.dot(hn, wo_ref[...],
                           preferred_element_type=jnp.float32) + bo_ref[...]


def _final(num, hprev, bias_l, W_out, b_out):
    full = lambda i: (0, 0)
    return pl.pallas_call(
        _final_body,
        grid=(NN // RB,),
        out_shape=jax.ShapeDtypeStruct((NN, DH), jnp.float32),
        in_specs=[
            pl.BlockSpec((2, RB, DH), lambda i: (0, i, 0)),
            pl.BlockSpec((RB, DH), lambda i: (i, 0)),
            pl.BlockSpec((1, DH), full),
            pl.BlockSpec((DH, DH), full),
            pl.BlockSpec((1, DH), full),
        ],
        out_specs=pl.BlockSpec((RB, DH), lambda i: (i, 0)),
    )(num, hprev, bias_l, W_out, b_out)


def kernel(x, edge_index, edge_attr, node_types, W_in, b_in, nt_table,
           et_table, Wl, bl, Wr, br, We, att, bias, W_out, b_out):
    loop = jnp.arange(NN, dtype=edge_index.dtype)
    npad = EP - (EE + NN)
    src2 = jnp.concatenate([edge_index[0], loop,
                            jnp.zeros((npad,), jnp.int32)])
    dst2 = jnp.concatenate([edge_index[1], loop,
                            jnp.full((npad,), JUNK_ROW, jnp.int32)])
    sel2 = jnp.concatenate([edge_attr, jnp.full((NN,), 3, jnp.int32),
                            jnp.zeros((npad,), jnp.int32)])
    dmin = jnp.minimum(dst2, NN - 1)
    src3 = jnp.concatenate([src2, src2 + NN]).reshape(2 * EP // CH, CH)
    dstg3 = jnp.concatenate([dmin, dmin + NN]).reshape(2 * EP // CH, CH)
    dst2d = dst2.reshape(EP // CH, CH)
    sel3 = jnp.concatenate([sel2, sel2 + 4])

    ntoh = (node_types[:, None] == jnp.arange(8)[None, :]).astype(jnp.float32)
    ntt8 = jnp.concatenate([nt_table, jnp.zeros((3, DH), jnp.float32)], axis=0)
    etp8 = jnp.concatenate([et_table, jnp.zeros((5, DH), jnp.float32)], axis=0)

    b_in2 = b_in.reshape(1, DH)
    bl2 = bl.reshape(NL, 1, DH)
    br2 = br.reshape(NL, 1, DH)
    bias2 = bias.reshape(NL, 1, DH)
    b_out2 = b_out.reshape(1, DH)

    counts = _edge_attr_hist(edge_attr.reshape(1250, 128))
    eetab = _eetab(etp8, counts, We)

    h, xlT, xrT = _prologue(x, W_in, b_in2, ntoh, ntt8,
                            Wl[0], bl2[0], Wr[0], br2[0])

    out = None
    for l in range(NL):
        num = _sc_edge_kernel(xlT.reshape(2 * NN, DH), xrT.reshape(2 * NN, DH),
                              src3, dstg3, dst2d, sel3, eetab[l], att[l])
        if l + 1 < NL:
            h, xlT, xrT = _combine(num, h, bias2[l],
                                   Wl[l + 1], bl2[l + 1], Wr[l + 1], br2[l + 1])
        else:
            out = _final(num, h, bias2[l], W_out, b_out2)
    return out


# speedup vs baseline: 1.1430x; 1.1323x over previous
import functools

import jax
import jax.numpy as jnp
from jax import lax
from jax.experimental import pallas as pl
from jax.experimental.pallas import tpu as pltpu
from jax.experimental.pallas import tpu_sc as plsc

NN = 10000
EE = 160000
DH = 128
NL = 3

NR = 10112
JUNK_ROW = 10008
CH = 64
IB = 8
CPB = IB * CH
NBLK = 21
NCHUNK = NBLK * IB
TPT = NCHUNK * CH
EP = 16 * TPT
RPT = NR // 16
RB = 1000


def _iotas():
    ri = lax.broadcasted_iota(jnp.int32, (DH, DH), 0)
    ci = lax.broadcasted_iota(jnp.int32, (DH, DH), 1)
    return ri, ci


def _perm_mats():
    ri, ci = _iotas()
    cc = ci % 32
    k = ci // 32
    feat = cc < 16
    p0 = (feat & (ri == 16 * k + cc)).astype(jnp.float32)
    p1 = (feat & (ri == 64 + 16 * k + cc)).astype(jnp.float32)
    return p0, p1


def _den_mat():
    ri, ci = _iotas()
    cc = ci % 32
    k = ci // 32
    return ((cc < 16) & (ri == 32 * k + 16 + cc)).astype(jnp.float32)


def _sel_mats():
    ri, ci = _iotas()
    rr = ri % 32
    k = ri // 32
    s0 = ((rr < 16) & (ci == 16 * k + rr)).astype(jnp.float32)
    s1 = ((rr < 16) & (ci == 64 + 16 * k + rr)).astype(jnp.float32)
    return s0, s1


def _hist_body(ea_ref, cnt_ref):
    ea = ea_ref[...]
    for t in range(3):
        cnt_ref[t] = jnp.sum((ea == t).astype(jnp.int32))


def _edge_attr_hist(ea2d):
    return pl.pallas_call(
        _hist_body,
        out_shape=jax.ShapeDtypeStruct((3,), jnp.int32),
        in_specs=[pl.BlockSpec(memory_space=pltpu.VMEM)],
        out_specs=pl.BlockSpec(memory_space=pltpu.SMEM),
    )(ea2d)


def _eetab_body(etp_ref, cnt_ref, we_ref, out_ref):
    etv = etp_ref[...]
    w = [cnt_ref[t].astype(jnp.float32) / float(EE) for t in range(3)]
    mean = w[0] * etv[0:1, :] + w[1] * etv[1:2, :] + w[2] * etv[2:3, :]
    rows = lax.broadcasted_iota(jnp.int32, (8, DH), 0)
    et_in = jnp.where(rows == 3, jnp.broadcast_to(mean, (8, DH)), etv)
    e = jnp.dot(et_in, we_ref[0], preferred_element_type=jnp.float32)
    p0, p1 = _perm_mats()
    e0 = jnp.dot(e, p0, preferred_element_type=jnp.float32)
    e1 = jnp.dot(e, p1, preferred_element_type=jnp.float32)
    out_ref[0] = jnp.concatenate([e0[0:4], e1[0:4]], axis=0)


def _eetab(etp, counts, We):
    return pl.pallas_call(
        _eetab_body,
        grid=(NL,),
        out_shape=jax.ShapeDtypeStruct((NL, 8, DH), jnp.float32),
        in_specs=[
            pl.BlockSpec((8, DH), lambda l: (0, 0)),
            pl.BlockSpec(memory_space=pltpu.SMEM),
            pl.BlockSpec((1, DH, DH), lambda l: (l, 0, 0)),
        ],
        out_specs=pl.BlockSpec((1, 8, DH), lambda l: (l, 0, 0)),
    )(etp, counts, We)


def _project(h, wl_ref, bl_ref, wr_ref, br_ref, xlT_ref, xrT_ref):
    p0, p1 = _perm_mats()
    xl = jnp.dot(h, wl_ref[...], preferred_element_type=jnp.float32) + bl_ref[...]
    xr = jnp.dot(h, wr_ref[...], preferred_element_type=jnp.float32) + br_ref[...]
    xlT_ref[0] = jnp.dot(xl, p0, preferred_element_type=jnp.float32)
    xlT_ref[1] = jnp.dot(xl, p1, preferred_element_type=jnp.float32)
    xrT_ref[0] = jnp.dot(xr, p0, preferred_element_type=jnp.float32)
    xrT_ref[1] = jnp.dot(xr, p1, preferred_element_type=jnp.float32)


def _prologue_body(x_ref, win_ref, bin_ref, ntoh_ref, ntt_ref,
                   wl_ref, bl_ref, wr_ref, br_ref,
                   h_ref, xlT_ref, xrT_ref):
    h = (jnp.dot(x_ref[...], win_ref[...], preferred_element_type=jnp.float32)
         + bin_ref[...]
         + jnp.dot(ntoh_ref[...], ntt_ref[...],
                   preferred_element_type=jnp.float32))
    h_ref[...] = h
    _project(h, wl_ref, bl_ref, wr_ref, br_ref, xlT_ref, xrT_ref)


def _prologue(x, W_in, b_in, ntoh, ntt, Wl0, bl0, Wr0, br0):
    full = lambda i: (0, 0)
    return pl.pallas_call(
        _prologue_body,
        grid=(NN // RB,),
        out_shape=[
            jax.ShapeDtypeStruct((NN, DH), jnp.float32),
            jax.ShapeDtypeStruct((2, NN, DH), jnp.float32),
            jax.ShapeDtypeStruct((2, NN, DH), jnp.float32),
        ],
        in_specs=[
            pl.BlockSpec((RB, DH), lambda i: (i, 0)),
            pl.BlockSpec((DH, DH), full),
            pl.BlockSpec((1, DH), full),
            pl.BlockSpec((RB, 8), lambda i: (i, 0)),
            pl.BlockSpec((8, DH), full),
            pl.BlockSpec((DH, DH), full),
            pl.BlockSpec((1, DH), full),
            pl.BlockSpec((DH, DH), full),
            pl.BlockSpec((1, DH), full),
        ],
        out_specs=[
            pl.BlockSpec((RB, DH), lambda i: (i, 0)),
            pl.BlockSpec((2, RB, DH), lambda i: (0, i, 0)),
            pl.BlockSpec((2, RB, DH), lambda i: (0, i, 0)),
        ],
    )(x, W_in, b_in, ntoh, ntt, Wl0, bl0, Wr0, br0)


_sc_mesh = plsc.VectorSubcoreMesh(core_axis_name="c", subcore_axis_name="s")


@functools.partial(
    pl.kernel,
    out_type=jax.ShapeDtypeStruct((2, NR, DH), jnp.float32),
    mesh=_sc_mesh,
    compiler_params=pltpu.CompilerParams(needs_layout_passes=False),
    scratch_types=[
        pltpu.VMEM((IB, CH), jnp.int32),
        pltpu.VMEM((IB, CH), jnp.int32),
        pltpu.VMEM((IB, CH), jnp.int32),
        pltpu.VMEM((CPB + 16,), jnp.int32),
        pltpu.VMEM((CH, DH), jnp.float32),
        pltpu.VMEM((CH, DH), jnp.float32),
        pltpu.VMEM((CH, DH), jnp.float32),
        pltpu.VMEM((CH, DH), jnp.float32),
        pltpu.VMEM((8, DH), jnp.float32),
        pltpu.VMEM((8, 16), jnp.float32),
        pltpu.VMEM_SHARED((NR, DH), jnp.float32),
        pltpu.SemaphoreType.DMA,
        pltpu.SemaphoreType.DMA,
        pltpu.SemaphoreType.DMA,
        pltpu.SemaphoreType.DMA,
    ],
)
def _sc_edge_kernel(xlT_hbm, xrT_hbm, src_hbm, dstg_hbm, dst_hbm, sel_hbm,
                    eetab_hbm, att_hbm, num_out,
                    isrc, idstg, idst, sel_v, xl0, xl1, xr0, xr1,
                    eetab_v, att_v, acc, sg0, sg1, ss0, ss1):
    cid = lax.axis_index("c")
    sid = lax.axis_index("s")
    row0 = sid * RPT
    cid4 = cid * 4
    gbase = cid * (EP // CH) + sid * (TPT // CH)
    rbase = sid * (TPT // CH)
    fbase = cid * EP + sid * TPT

    pltpu.sync_copy(eetab_hbm, eetab_v)
    pltpu.sync_copy(att_hbm, att_v)
    att_k = [att_v[k4, :] for k4 in range(4)]
    att_k = [jnp.where(cid4 > 0, att_v[4 + k4, :], att_k[k4])
             for k4 in range(4)]

    xlb = [xl0, xl1]
    xrb = [xr0, xr1]
    sg = [sg0, sg1]
    ss = [ss0, ss1]

    zl16 = jnp.zeros((16,), jnp.float32)

    def zrow(r, _):
        for h in range(8):
            xl0[r, pl.ds(h * 16, 16)] = zl16
        return 0

    lax.fori_loop(0, CH, zrow, 0)
    for k in range(RPT // CH):
        pltpu.sync_copy(xl0, acc.at[pl.ds(row0 + k * CH, CH)])
    tail = RPT - (RPT // CH) * CH
    if tail:
        pltpu.sync_copy(xl0.at[pl.ds(0, tail)],
                        acc.at[pl.ds(row0 + (RPT // CH) * CH, tail)])
    plsc.subcore_barrier()

    def run_edges(b, c):
        @plsc.parallel_loop(0, CH, unroll=4)
        def edge_body(j):
            s = sel_v[pl.ds(c * CH + j, 16)][0] + cid4
            for k4 in range(4):
                fsl = pl.ds(k4 * 32, 16)
                psl = pl.ds(k4 * 32 + 16, 16)
                xf = xlb[b][j, fsl]
                m = xf + xrb[b][j, fsl] + eetab_v[s, fsl]
                m = jnp.where(m >= 0.0, m, m * 0.2)
                a = jnp.sum(m * att_k[k4])
                pv = jnp.exp(jnp.broadcast_to(a, (16,)))
                xlb[b][j, fsl] = pv * xf
                xlb[b][j, psl] = pv

    def start_gather(c, b):
        g1 = pltpu.async_copy(xlT_hbm.at[isrc.at[c]], xlb[b], sg[b])
        g2 = pltpu.async_copy(xrT_hbm.at[idstg.at[c]], xrb[b], sg[b])
        return g1, g2

    def block_body(blk, _):
        pltpu.sync_copy(src_hbm.at[pl.ds(gbase + blk * IB, IB)], isrc)
        pltpu.sync_copy(dstg_hbm.at[pl.ds(gbase + blk * IB, IB)], idstg)
        pltpu.sync_copy(dst_hbm.at[pl.ds(rbase + blk * IB, IB)], idst)
        pltpu.sync_copy(sel_hbm.at[pl.ds(fbase + blk * CPB, CPB)],
                        sel_v.at[pl.ds(0, CPB)])
        gd = start_gather(0, 0)
        sd = [None, None]
        for c in range(IB):
            b = c % 2
            b1 = 1 - b
            if c + 1 < IB:
                if sd[b1] is not None:
                    sd[b1].wait()
                gd_next = start_gather(c + 1, b1)
            gd[0].wait()
            gd[1].wait()
            run_edges(b, c)
            sd[b] = pltpu.async_copy(xlb[b], acc.at[idst.at[c]], ss[b],
                                     add=True)
            if c + 1 < IB:
                gd = gd_next
        sd[0].wait()
        sd[1].wait()
        return 0

    lax.fori_loop(0, NBLK, block_body, 0)
    plsc.subcore_barrier()

    pltpu.sync_copy(acc.at[pl.ds(row0, RPT)],
                    num_out.at[cid, pl.ds(row0, RPT)])


def _normalize(num_ref, hprev_ref, bias_ref):
    nu0 = num_ref[0]
    nu1 = num_ref[1]
    d = _den_mat()
    s0, s1 = _sel_mats()
    der0 = jnp.dot(nu0, d, preferred_element_type=jnp.float32)
    der1 = jnp.dot(nu1, d, preferred_element_type=jnp.float32)
    r0 = nu0 / (der0 + 1e-16)
    r1 = nu1 / (der1 + 1e-16)
    o = (jnp.dot(r0, s0, preferred_element_type=jnp.float32)
         + jnp.dot(r1, s1, preferred_element_type=jnp.float32)
         + bias_ref[...])
    return jnp.maximum(o, 0.0) + hprev_ref[...]


def _combine_body(num_ref, hprev_ref, bias_ref,
                  wl_ref, bl_ref, wr_ref, br_ref,
                  h_ref, xlT_ref, xrT_ref):
    hn = _normalize(num_ref, hprev_ref, bias_ref)
    h_ref[...] = hn
    _project(hn, wl_ref, bl_ref, wr_ref, br_ref, xlT_ref, xrT_ref)


def _combine(num, hprev, bias_l, Wln, bln, Wrn, brn):
    full = lambda i: (0, 0)
    return pl.pallas_call(
        _combine_body,
        grid=(NN // RB,),
        out_shape=[
            jax.ShapeDtypeStruct((NN, DH), jnp.float32),
            jax.ShapeDtypeStruct((2, NN, DH), jnp.float32),
            jax.ShapeDtypeStruct((2, NN, DH), jnp.float32),
        ],
        in_specs=[
            pl.BlockSpec((2, RB, DH), lambda i: (0, i, 0)),
            pl.BlockSpec((RB, DH), lambda i: (i, 0)),
            pl.BlockSpec((1, DH), full),
            pl.BlockSpec((DH, DH), full),
            pl.BlockSpec((1, DH), full),
            pl.BlockSpec((DH, DH), full),
            pl.BlockSpec((1, DH), full),
        ],
        out_specs=[
            pl.BlockSpec((RB, DH), lambda i: (i, 0)),
            pl.BlockSpec((2, RB, DH), lambda i: (0, i, 0)),
            pl.BlockSpec((2, RB, DH), lambda i: (0, i, 0)),
        ],
    )(num, hprev, bias_l, Wln, bln, Wrn, brn)


def _final_body(num_ref, hprev_ref, bias_ref, wo_ref, bo_ref, out_ref):
    hn = _normalize(num_ref, hprev_ref, bias_ref)
    out_ref[...] = jnp.dot(hn, wo_ref[...],
                           preferred_element_type=jnp.float32) + bo_ref[...]


def _final(num, hprev, bias_l, W_out, b_out):
    full = lambda i: (0, 0)
    return pl.pallas_call(
        _final_body,
        grid=(NN // RB,),
        out_shape=jax.ShapeDtypeStruct((NN, DH), jnp.float32),
        in_specs=[
            pl.BlockSpec((2, RB, DH), lambda i: (0, i, 0)),
            pl.BlockSpec((RB, DH), lambda i: (i, 0)),
            pl.BlockSpec((1, DH), full),
            pl.BlockSpec((DH, DH), full),
            pl.BlockSpec((1, DH), full),
        ],
        out_specs=pl.BlockSpec((RB, DH), lambda i: (i, 0)),
    )(num, hprev, bias_l, W_out, b_out)


def kernel(x, edge_index, edge_attr, node_types, W_in, b_in, nt_table,
           et_table, Wl, bl, Wr, br, We, att, bias, W_out, b_out):
    loop = jnp.arange(NN, dtype=edge_index.dtype)
    npad = EP - (EE + NN)
    src2 = jnp.concatenate([edge_index[0], loop,
                            jnp.zeros((npad,), jnp.int32)])
    dst2 = jnp.concatenate([edge_index[1], loop,
                            jnp.full((npad,), JUNK_ROW, jnp.int32)])
    sel2 = jnp.concatenate([edge_attr, jnp.full((NN,), 3, jnp.int32),
                            jnp.zeros((npad,), jnp.int32)])
    dmin = jnp.minimum(dst2, NN - 1)
    src3 = jnp.concatenate([src2, src2 + NN]).reshape(2 * EP // CH, CH)
    dstg3 = jnp.concatenate([dmin, dmin + NN]).reshape(2 * EP // CH, CH)
    dst2d = dst2.reshape(EP // CH, CH)
    sel3 = jnp.concatenate([sel2, sel2 + 4])

    ntoh = (node_types[:, None] == jnp.arange(8)[None, :]).astype(jnp.float32)
    ntt8 = jnp.concatenate([nt_table, jnp.zeros((3, DH), jnp.float32)], axis=0)
    etp8 = jnp.concatenate([et_table, jnp.zeros((5, DH), jnp.float32)], axis=0)

    b_in2 = b_in.reshape(1, DH)
    bl2 = bl.reshape(NL, 1, DH)
    br2 = br.reshape(NL, 1, DH)
    bias2 = bias.reshape(NL, 1, DH)
    b_out2 = b_out.reshape(1, DH)

    counts = _edge_attr_hist(edge_attr.reshape(1250, 128))
    eetab = _eetab(etp8, counts, We)

    h, xlT, xrT = _prologue(x, W_in, b_in2, ntoh, ntt8,
                            Wl[0], bl2[0], Wr[0], br2[0])

    out = None
    for l in range(NL):
        num = _sc_edge_kernel(xlT.reshape(2 * NN, DH), xrT.reshape(2 * NN, DH),
                              src3, dstg3, dst2d, sel3, eetab[l], att[l])
        if l + 1 < NL:
            h, xlT, xrT = _combine(num, h, bias2[l],
                                   Wl[l + 1], bl2[l + 1], Wr[l + 1], br2[l + 1])
        else:
            out = _final(num, h, bias2[l], W_out, b_out2)
    return out
